# baseline (speedup 1.0000x reference)
import functools

import jax
import jax.numpy as jnp
from jax.experimental import pallas as pl
from jax.experimental.pallas import tpu as pltpu

E = 8
H = 8192
FF = 16384
T = 32
BF = 256


def _topk2_dense_weights(logits):
    logits = logits.astype(jnp.float32)
    m = jnp.max(logits, axis=-1, keepdims=True)
    ex = jnp.exp(logits - m)
    p = ex / jnp.sum(ex, axis=-1, keepdims=True)

    ii = jax.lax.broadcasted_iota(jnp.int32, p.shape, 1)
    big = jnp.int32(E)
    m1 = jnp.max(p, axis=-1, keepdims=True)
    idx1 = jnp.min(jnp.where(p == m1, ii, big), axis=-1, keepdims=True)
    mask1 = ii == idx1
    p2 = jnp.where(mask1, -jnp.inf, p)
    m2 = jnp.max(p2, axis=-1, keepdims=True)
    idx2 = jnp.min(jnp.where(p2 == m2, ii, big), axis=-1, keepdims=True)
    mask2 = ii == idx2
    return jnp.where(mask1 | mask2, p, 0.0)


def _moe_kernel(x_ref, gw_ref, w1_ref, w3_ref, w2_ref, y_ref, wt_ref):
    e = pl.program_id(0)
    f = pl.program_id(1)

    @pl.when((e == 0) & (f == 0))
    def _init():
        logits = jax.lax.dot_general(
            x_ref[...], gw_ref[...], (((1,), (1,)), ((), ())),
            preferred_element_type=jnp.float32)
        wt_ref[...] = _topk2_dense_weights(logits)
        y_ref[...] = jnp.zeros_like(y_ref)

    x = x_ref[...].astype(jnp.bfloat16)
    w1 = w1_ref[0].astype(jnp.bfloat16)
    w3 = w3_ref[0].astype(jnp.bfloat16)
    w2 = w2_ref[0].astype(jnp.bfloat16)

    a = jax.lax.dot_general(x, w1, (((1,), (1,)), ((), ())),
                            preferred_element_type=jnp.float32)
    b = jax.lax.dot_general(x, w3, (((1,), (1,)), ((), ())),
                            preferred_element_type=jnp.float32)
    gelu_a = a * 0.5 * (1.0 + jax.lax.erf(a * 0.7071067811865476))
    h = gelu_a * b

    lane = jax.lax.broadcasted_iota(jnp.int32, (T, E), 1)
    wcol = jnp.sum(jnp.where(lane == e, wt_ref[...], 0.0), axis=1,
                   keepdims=True)
    h = h * wcol

    y_ref[...] += jax.lax.dot_general(
        h.astype(jnp.bfloat16), w2, (((1,), (1,)), ((), ())),
        preferred_element_type=jnp.float32)


@functools.partial(jax.jit, static_argnames=())
def _moe(x2d, gate_w, w1, w2, w3):
    grid = (E, FF // BF)
    y = pl.pallas_call(
        _moe_kernel,
        grid=grid,
        in_specs=[
            pl.BlockSpec((T, H), lambda e, f: (0, 0)),
            pl.BlockSpec((E, H), lambda e, f: (0, 0)),
            pl.BlockSpec((1, BF, H), lambda e, f: (e, f, 0)),
            pl.BlockSpec((1, BF, H), lambda e, f: (e, f, 0)),
            pl.BlockSpec((1, H, BF), lambda e, f: (e, 0, f)),
        ],
        out_specs=pl.BlockSpec((T, H), lambda e, f: (0, 0)),
        out_shape=jax.ShapeDtypeStruct((T, H), jnp.float32),
        scratch_shapes=[pltpu.VMEM((T, E), jnp.float32)],
        compiler_params=pltpu.CompilerParams(
            dimension_semantics=("arbitrary", "arbitrary")),
    )(x2d, gate_w, w1, w3, w2)
    return y


def kernel(x, gate_w, w1, w2, w3):
    x2d = x.reshape(T, H)
    y = _moe(x2d, gate_w, w1, w2, w3)
    return y.reshape(x.shape)


# speedup vs baseline: 1.0060x; 1.0060x over previous
import functools

import jax
import jax.numpy as jnp
from jax.experimental import pallas as pl
from jax.experimental.pallas import tpu as pltpu

E = 8
H = 8192
FF = 16384
T = 32
BF = 256
NBF = FF // BF
BH = 128


def _topk2_dense_weights(logits):
    logits = logits.astype(jnp.float32)
    m = jnp.max(logits, axis=-1, keepdims=True)
    ex = jnp.exp(logits - m)
    p = ex / jnp.sum(ex, axis=-1, keepdims=True)

    ii = jax.lax.broadcasted_iota(jnp.int32, p.shape, 1)
    big = jnp.int32(E)
    m1 = jnp.max(p, axis=-1, keepdims=True)
    idx1 = jnp.min(jnp.where(p == m1, ii, big), axis=-1, keepdims=True)
    mask1 = ii == idx1
    p2 = jnp.where(mask1, -jnp.inf, p)
    m2 = jnp.max(p2, axis=-1, keepdims=True)
    idx2 = jnp.min(jnp.where(p2 == m2, ii, big), axis=-1, keepdims=True)
    mask2 = ii == idx2
    return jnp.where(mask1 | mask2, p, 0.0)


def _moe_kernel(x_ref, gw_ref, w1_ref, w3_ref, w2_ref, y_ref, wt_ref, h_ref):
    e = pl.program_id(0)
    f = pl.program_id(1)

    @pl.when((e == 0) & (f == 0))
    def _init():
        logits = jax.lax.dot_general(
            x_ref[...], gw_ref[...], (((1,), (1,)), ((), ())),
            preferred_element_type=jnp.float32)
        wt_ref[...] = _topk2_dense_weights(logits)
        y_ref[...] = jnp.zeros_like(y_ref)

    @pl.when(e < E)
    def _up():
        x = x_ref[...].astype(jnp.bfloat16)
        w1 = w1_ref[0].astype(jnp.bfloat16)
        w3 = w3_ref[0].astype(jnp.bfloat16)
        a = jax.lax.dot_general(x, w1, (((1,), (1,)), ((), ())),
                                preferred_element_type=jnp.float32)
        b = jax.lax.dot_general(x, w3, (((1,), (1,)), ((), ())),
                                preferred_element_type=jnp.float32)
        gelu_a = a * 0.5 * (1.0 + jax.lax.erf(a * 0.7071067811865476))
        h = gelu_a * b
        lane = jax.lax.broadcasted_iota(jnp.int32, (T, E), 1)
        wcol = jnp.sum(jnp.where(lane == e, wt_ref[...], 0.0), axis=1,
                       keepdims=True)
        h_ref[e % 2, :, pl.ds(f * BF, BF)] = (h * wcol).astype(jnp.bfloat16)

    @pl.when(e > 0)
    def _down():
        hprev = h_ref[(e + 1) % 2]
        w2 = w2_ref[0].astype(jnp.bfloat16)
        yblk = jax.lax.dot_general(hprev, w2, (((1,), (1,)), ((), ())),
                                   preferred_element_type=jnp.float32)
        y_ref[:, pl.ds(f * BH, BH)] += yblk


def _w13_index(e, f):
    ec = jnp.minimum(e, E - 1)
    fc = jnp.where(e == E, NBF - 1, f)
    return (ec, fc, 0)


def _w2_index(e, f):
    ec = jnp.maximum(e - 1, 0)
    fc = jnp.where(e == 0, 0, f)
    return (ec, fc, 0)


@jax.jit
def _moe(x2d, gate_w, w1, w2, w3):
    grid = (E + 1, NBF)
    y = pl.pallas_call(
        _moe_kernel,
        grid=grid,
        in_specs=[
            pl.BlockSpec((T, H), lambda e, f: (0, 0)),
            pl.BlockSpec((E, H), lambda e, f: (0, 0)),
            pl.BlockSpec((1, BF, H), _w13_index),
            pl.BlockSpec((1, BF, H), _w13_index),
            pl.BlockSpec((1, BH, FF), _w2_index),
        ],
        out_specs=pl.BlockSpec((T, H), lambda e, f: (0, 0)),
        out_shape=jax.ShapeDtypeStruct((T, H), jnp.float32),
        scratch_shapes=[
            pltpu.VMEM((T, E), jnp.float32),
            pltpu.VMEM((2, T, FF), jnp.bfloat16),
        ],
        compiler_params=pltpu.CompilerParams(
            dimension_semantics=("arbitrary", "arbitrary")),
    )(x2d, gate_w, w1, w3, w2)
    return y


def kernel(x, gate_w, w1, w2, w3):
    x2d = x.reshape(T, H)
    y = _moe(x2d, gate_w, w1, w2, w3)
    return y.reshape(x.shape)
